# baseline (speedup 1.0000x reference)
import jax
import jax.numpy as jnp
from jax import lax
from jax.experimental import pallas as pl
from jax.experimental.pallas import tpu as pltpu
from jax.experimental.pallas import tpu_sc as plsc

HW = 128 // 2

B, L, H = 1024, 200, 128
N = B * L
EPS = 1e-12

NC, NS, LANES = 2, 16, 16
NW = NC * NS
NSLICE = 4
SL = N // NSLICE
TPW = SL // NW
CHUNK = 80
NCH = TPW // CHUNK
NBUF = 2

TOK_BLK = 2048
BLKS = SL // TOK_BLK
SEG_V = 16


def _sc_body(dids, pids, dtab, ptab, out,
             idxd_v, idxp_v,
             bufd0, bufd1, bufp0, bufp1, bufo0, bufo1,
             semd0, semd1, semp0, semp1, semo0, semo1):
    bufd = (bufd0, bufd1)
    bufp = (bufp0, bufp1)
    bufo = (bufo0, bufo1)
    semd = (semd0, semd1)
    semp = (semp0, semp1)
    semo = (semo0, semo1)

    wid = lax.axis_index("s") * NC + lax.axis_index("c")
    pltpu.sync_copy(dids.at[wid], idxd_v)
    pltpu.sync_copy(pids.at[wid], idxp_v)

    def start_gathers(j, b):
        pltpu.make_async_copy(dtab.at[idxd_v.at[j]], bufd[b], semd[b]).start()
        pltpu.make_async_copy(ptab.at[idxp_v.at[j]], bufp[b], semp[b]).start()

    for b in range(NBUF):
        start_gathers(b, b)

    def outer(i, carry):
        j0 = i * NBUF
        for b in range(NBUF):
            j = j0 + b
            pltpu.make_async_copy(dtab.at[idxd_v.at[j]], bufd[b], semd[b]).wait()
            pltpu.make_async_copy(ptab.at[idxp_v.at[j]], bufp[b], semp[b]).wait()

            @pl.when(i > 0)
            def _():
                pltpu.make_async_copy(
                    bufo[b], out.at[pl.ds(0, CHUNK)], semo[b]).wait()

            def tok_body(t, c2):
                for hp in range(H // (2 * LANES)):
                    sla = pl.ds(2 * hp * LANES, LANES)
                    slb = pl.ds((2 * hp + 1) * LANES, LANES)
                    va = bufd[b][t, sla] + bufp[b][t, sla]
                    vb = bufd[b][t, slb] + bufp[b][t, slb]
                    wa = (plsc.bitcast(va, jnp.int32) >> 16) & jnp.int32(0xFFFF)
                    wb = plsc.bitcast(vb, jnp.int32) & jnp.int32(-65536)
                    bufo[b][t, pl.ds(hp * LANES, LANES)] = wa | wb
                return c2

            lax.fori_loop(0, CHUNK, tok_body, 0, unroll=2)

            @pl.when(j + NBUF < NCH)
            def _():
                start_gathers(j + NBUF, b)

            pltpu.make_async_copy(
                bufo[b], out.at[pl.ds(wid * TPW + j * CHUNK, CHUNK)],
                semo[b]).start()
        return carry

    lax.fori_loop(0, NCH // NBUF, outer, 0)

    for b in range(NBUF):
        pltpu.make_async_copy(bufo[b], out.at[pl.ds(0, CHUNK)], semo[b]).wait()


def _sc_gather_sum(dids3, pids3, delays_table, posi_table):
    mesh = plsc.VectorSubcoreMesh(core_axis_name="c", subcore_axis_name="s")
    f = pl.kernel(
        _sc_body,
        mesh=mesh,
        compiler_params=pltpu.CompilerParams(needs_layout_passes=False),
        out_type=jax.ShapeDtypeStruct((SL, HW), jnp.int32),
        scratch_types=[
            pltpu.VMEM((NCH, CHUNK), jnp.int32),
            pltpu.VMEM((NCH, CHUNK), jnp.int32),
            pltpu.VMEM((CHUNK, H), jnp.float32),
            pltpu.VMEM((CHUNK, H), jnp.float32),
            pltpu.VMEM((CHUNK, H), jnp.float32),
            pltpu.VMEM((CHUNK, H), jnp.float32),
            pltpu.VMEM((CHUNK, HW), jnp.int32),
            pltpu.VMEM((CHUNK, HW), jnp.int32),
            pltpu.SemaphoreType.DMA,
            pltpu.SemaphoreType.DMA,
            pltpu.SemaphoreType.DMA,
            pltpu.SemaphoreType.DMA,
            pltpu.SemaphoreType.DMA,
            pltpu.SemaphoreType.DMA,
        ],
    )
    return f(dids3, pids3, delays_table, posi_table)


def _ln_math(word, scrw, ids, segtab, gamma, beta):
    oh = (ids[:, None] == lax.broadcasted_iota(jnp.int32, (TOK_BLK, SEG_V), 1)
          ).astype(jnp.float32)
    segrows = jnp.dot(oh, segtab, preferred_element_type=jnp.float32)
    lo = jax.lax.bitcast_convert_type(scrw << 16, jnp.float32)
    hi = jax.lax.bitcast_convert_type(scrw & jnp.int32(-65536), jnp.float32)
    scr = jnp.stack([lo.reshape(TOK_BLK, H // 32, LANES),
                     hi.reshape(TOK_BLK, H // 32, LANES)],
                    axis=2).reshape(TOK_BLK, H)
    s = word + scr + segrows
    mean = jnp.mean(s, axis=-1, keepdims=True)
    c = s - mean
    var = jnp.mean(c * c, axis=-1, keepdims=True)
    return c * jax.lax.rsqrt(var + EPS) * gamma + beta


def _ln_body0(word_ref, scr_ref, seg_ref, segtab_ref, gamma_ref, beta_ref,
              out_ref):
    out_ref[...] = _ln_math(word_ref[...], scr_ref[...], seg_ref[0, 0, :],
                            segtab_ref[...], gamma_ref[...], beta_ref[...])


def _ln_body_acc(acc_ref, word_ref, scr_ref, seg_ref, segtab_ref, gamma_ref,
                 beta_ref, out_ref):
    del acc_ref
    out_ref[...] = _ln_math(word_ref[...], scr_ref[...], seg_ref[0, 0, :],
                            segtab_ref[...], gamma_ref[...], beta_ref[...])


def _tc_ln_slice(s, prev, word_flat, scratch_s, seg3, seg_table, g2, b2):
    common_in = [
        pl.BlockSpec((TOK_BLK, H), lambda i, s=s: (s * BLKS + i, 0)),
        pl.BlockSpec((TOK_BLK, HW), lambda i: (i, 0)),
        pl.BlockSpec((1, 1, TOK_BLK), lambda i, s=s: (s * BLKS + i, 0, 0)),
        pl.BlockSpec((SEG_V, H), lambda i: (0, 0)),
        pl.BlockSpec((1, H), lambda i: (0, 0)),
        pl.BlockSpec((1, H), lambda i: (0, 0)),
    ]
    out_spec = pl.BlockSpec((TOK_BLK, H), lambda i, s=s: (s * BLKS + i, 0))
    if prev is None:
        return pl.pallas_call(
            _ln_body0,
            grid=(BLKS,),
            in_specs=common_in,
            out_specs=out_spec,
            out_shape=jax.ShapeDtypeStruct((N, H), jnp.float32),
        )(word_flat, scratch_s, seg3, seg_table, g2, b2)
    return pl.pallas_call(
        _ln_body_acc,
        grid=(BLKS,),
        in_specs=[pl.BlockSpec((8, H), lambda i: (0, 0))] + common_in,
        out_specs=out_spec,
        out_shape=jax.ShapeDtypeStruct((N, H), jnp.float32),
        input_output_aliases={0: 0},
    )(prev, word_flat, scratch_s, seg3, seg_table, g2, b2)


def kernel(word_ids, delays_ids, seg_ids, posi_ids, seg_table, delays_table,
           posi_table, ln_gamma, ln_beta):
    dids4 = delays_ids.reshape(NSLICE, NW, NCH, CHUNK).astype(jnp.int32)
    pids4 = posi_ids.reshape(NSLICE, NW, NCH, CHUNK).astype(jnp.int32)
    seg3 = seg_ids.reshape(N // TOK_BLK, 1, TOK_BLK).astype(jnp.int32)
    word_flat = word_ids.reshape(N, H)
    g2 = ln_gamma.reshape(1, H)
    b2 = ln_beta.reshape(1, H)

    scratches = [
        _sc_gather_sum(dids4[s], pids4[s], delays_table, posi_table)
        for s in range(NSLICE)
    ]
    out = None
    for s in range(NSLICE):
        out = _tc_ln_slice(s, out, word_flat, scratches[s], seg3, seg_table,
                           g2, b2)
    return out.reshape(B, L, H)


# speedup vs baseline: 3.1987x; 3.1987x over previous
import jax
import jax.numpy as jnp
from jax import lax
from jax.experimental import pallas as pl
from jax.experimental.pallas import tpu as pltpu
from jax.experimental.pallas import tpu_sc as plsc

HW = 128 // 2

B, L, H = 1024, 200, 128
N = B * L
EPS = 1e-12

NC, NS, LANES = 2, 16, 16
NW = NC * NS
NSLICE = 4
SL = N // NSLICE
TPW = SL // NW
CHUNK = 80
NCH = TPW // CHUNK
NBUF = 2

TOK_BLK = 2048
BLKS = SL // TOK_BLK
SEG_V = 16


def _sc_body(dids, pids, dtab, ptab, out,
             idxd_v, idxp_v,
             bufd0, bufd1, bufp0, bufp1, bufo0, bufo1,
             semd0, semd1, semp0, semp1, semo0, semo1):
    bufd = (bufd0, bufd1)
    bufp = (bufp0, bufp1)
    bufo = (bufo0, bufo1)
    semd = (semd0, semd1)
    semp = (semp0, semp1)
    semo = (semo0, semo1)

    wid = lax.axis_index("s") * NC + lax.axis_index("c")
    pltpu.sync_copy(dids.at[wid], idxd_v)
    pltpu.sync_copy(pids.at[wid], idxp_v)

    def start_gathers(j, b):
        pltpu.make_async_copy(dtab.at[idxd_v.at[j]], bufd[b], semd[b]).start()
        pltpu.make_async_copy(ptab.at[idxp_v.at[j]], bufp[b], semp[b]).start()

    for b in range(NBUF):
        start_gathers(b, b)

    def outer(i, carry):
        j0 = i * NBUF
        for b in range(NBUF):
            j = j0 + b
            pltpu.make_async_copy(dtab.at[idxd_v.at[j]], bufd[b], semd[b]).wait()
            pltpu.make_async_copy(ptab.at[idxp_v.at[j]], bufp[b], semp[b]).wait()

            @pl.when(i > 0)
            def _():
                pltpu.make_async_copy(
                    bufo[b], out.at[pl.ds(0, CHUNK)], semo[b]).wait()

            def tok_body(t, c2):
                for hp in range(H // (2 * LANES)):
                    sla = pl.ds(hp * LANES, LANES)
                    slb = pl.ds((hp + 4) * LANES, LANES)
                    va = bufd[b][t, sla] + bufp[b][t, sla]
                    vb = bufd[b][t, slb] + bufp[b][t, slb]
                    wa = (plsc.bitcast(va, jnp.int32) >> 16) & jnp.int32(0xFFFF)
                    wb = plsc.bitcast(vb, jnp.int32) & jnp.int32(-65536)
                    bufo[b][t, pl.ds(hp * LANES, LANES)] = wa | wb
                return c2

            lax.fori_loop(0, CHUNK, tok_body, 0, unroll=2)

            @pl.when(j + NBUF < NCH)
            def _():
                start_gathers(j + NBUF, b)

            pltpu.make_async_copy(
                bufo[b], out.at[pl.ds(wid * TPW + j * CHUNK, CHUNK)],
                semo[b]).start()
        return carry

    lax.fori_loop(0, NCH // NBUF, outer, 0)

    for b in range(NBUF):
        pltpu.make_async_copy(bufo[b], out.at[pl.ds(0, CHUNK)], semo[b]).wait()


def _sc_gather_sum(dids3, pids3, delays_table, posi_table):
    mesh = plsc.VectorSubcoreMesh(core_axis_name="c", subcore_axis_name="s")
    f = pl.kernel(
        _sc_body,
        mesh=mesh,
        compiler_params=pltpu.CompilerParams(needs_layout_passes=False),
        out_type=jax.ShapeDtypeStruct((SL, HW), jnp.int32),
        scratch_types=[
            pltpu.VMEM((NCH, CHUNK), jnp.int32),
            pltpu.VMEM((NCH, CHUNK), jnp.int32),
            pltpu.VMEM((CHUNK, H), jnp.float32),
            pltpu.VMEM((CHUNK, H), jnp.float32),
            pltpu.VMEM((CHUNK, H), jnp.float32),
            pltpu.VMEM((CHUNK, H), jnp.float32),
            pltpu.VMEM((CHUNK, HW), jnp.int32),
            pltpu.VMEM((CHUNK, HW), jnp.int32),
            pltpu.SemaphoreType.DMA,
            pltpu.SemaphoreType.DMA,
            pltpu.SemaphoreType.DMA,
            pltpu.SemaphoreType.DMA,
            pltpu.SemaphoreType.DMA,
            pltpu.SemaphoreType.DMA,
        ],
    )
    return f(dids3, pids3, delays_table, posi_table)


def _ln_math(word, scrw, ids, segtab, gamma, beta):
    oh = (ids[:, None] == lax.broadcasted_iota(jnp.int32, (TOK_BLK, SEG_V), 1)
          ).astype(jnp.float32)
    segrows = jnp.dot(oh, segtab, preferred_element_type=jnp.float32)
    lo = jax.lax.bitcast_convert_type(scrw << 16, jnp.float32)
    hi = jax.lax.bitcast_convert_type(scrw & jnp.int32(-65536), jnp.float32)
    scr = jnp.concatenate([lo, hi], axis=-1)
    s = word + scr + segrows
    mean = jnp.mean(s, axis=-1, keepdims=True)
    c = s - mean
    var = jnp.mean(c * c, axis=-1, keepdims=True)
    return c * jax.lax.rsqrt(var + EPS) * gamma + beta


def _ln_body0(word_ref, scr_ref, seg_ref, segtab_ref, gamma_ref, beta_ref,
              out_ref):
    out_ref[...] = _ln_math(word_ref[...], scr_ref[...], seg_ref[0, 0, :],
                            segtab_ref[...], gamma_ref[...], beta_ref[...])


def _ln_body_acc(acc_ref, word_ref, scr_ref, seg_ref, segtab_ref, gamma_ref,
                 beta_ref, out_ref):
    del acc_ref
    out_ref[...] = _ln_math(word_ref[...], scr_ref[...], seg_ref[0, 0, :],
                            segtab_ref[...], gamma_ref[...], beta_ref[...])


def _tc_ln_slice(s, prev, word_flat, scratch_s, seg3, seg_table, g2, b2):
    common_in = [
        pl.BlockSpec((TOK_BLK, H), lambda i, s=s: (s * BLKS + i, 0)),
        pl.BlockSpec((TOK_BLK, HW), lambda i: (i, 0)),
        pl.BlockSpec((1, 1, TOK_BLK), lambda i, s=s: (s * BLKS + i, 0, 0)),
        pl.BlockSpec((SEG_V, H), lambda i: (0, 0)),
        pl.BlockSpec((1, H), lambda i: (0, 0)),
        pl.BlockSpec((1, H), lambda i: (0, 0)),
    ]
    out_spec = pl.BlockSpec((TOK_BLK, H), lambda i, s=s: (s * BLKS + i, 0))
    if prev is None:
        return pl.pallas_call(
            _ln_body0,
            grid=(BLKS,),
            in_specs=common_in,
            out_specs=out_spec,
            out_shape=jax.ShapeDtypeStruct((N, H), jnp.float32),
        )(word_flat, scratch_s, seg3, seg_table, g2, b2)
    return pl.pallas_call(
        _ln_body_acc,
        grid=(BLKS,),
        in_specs=[pl.BlockSpec((8, H), lambda i: (0, 0))] + common_in,
        out_specs=out_spec,
        out_shape=jax.ShapeDtypeStruct((N, H), jnp.float32),
        input_output_aliases={0: 0},
    )(prev, word_flat, scratch_s, seg3, seg_table, g2, b2)


def kernel(word_ids, delays_ids, seg_ids, posi_ids, seg_table, delays_table,
           posi_table, ln_gamma, ln_beta):
    dids4 = delays_ids.reshape(NSLICE, NW, NCH, CHUNK).astype(jnp.int32)
    pids4 = posi_ids.reshape(NSLICE, NW, NCH, CHUNK).astype(jnp.int32)
    seg3 = seg_ids.reshape(N // TOK_BLK, 1, TOK_BLK).astype(jnp.int32)
    word_flat = word_ids.reshape(N, H)
    g2 = ln_gamma.reshape(1, H)
    b2 = ln_beta.reshape(1, H)

    scratches = [
        _sc_gather_sum(dids4[s], pids4[s], delays_table, posi_table)
        for s in range(NSLICE)
    ]
    out = None
    for s in range(NSLICE):
        out = _tc_ln_slice(s, out, word_flat, scratches[s], seg3, seg_table,
                           g2, b2)
    return out.reshape(B, L, H)
